# baseline (speedup 1.0000x reference)
import functools

import jax
import jax.numpy as jnp
from jax import lax
from jax.experimental import pallas as pl
from jax.experimental.pallas import tpu as pltpu
from jax.experimental.pallas import tpu_sc as plsc

_NUM_EMB = 1_000_000
_DIM = 32
_CHUNK = 128


def _row_mean_body(x_ref, o_ref):
    o_ref[...] = jnp.sum(x_ref[...], axis=1).reshape(1, 1, -1) * (1.0 / _DIM)


def _row_means(table):
    rows = table.shape[0]
    blk = 8_000
    out = pl.pallas_call(
        _row_mean_body,
        grid=(rows // blk,),
        in_specs=[pl.BlockSpec((blk, _DIM), lambda i: (i, 0))],
        out_specs=pl.BlockSpec((1, 1, blk), lambda i: (i, 0, 0)),
        out_shape=jax.ShapeDtypeStruct((rows // blk, 1, blk), jnp.float32),
    )(table)
    return out.reshape(rows)


def _gather_means(means, idx1d):
    info = plsc.get_sparse_core_info()
    nw = info.num_cores * info.num_subcores
    n = idx1d.shape[0]
    npw = n // nw
    chunks = npw // _CHUNK
    k = 10
    mesh = plsc.VectorSubcoreMesh(core_axis_name="c", subcore_axis_name="s")

    @functools.partial(
        pl.kernel, mesh=mesh,
        out_type=jax.ShapeDtypeStruct((n,), jnp.float32),
        scratch_types=[
            pltpu.VMEM((npw,), jnp.int32),
            pltpu.VMEM((npw,), jnp.float32),
            pltpu.SemaphoreType.DMA,
        ],
    )
    def gather_kernel(means_hbm, idx_hbm, out_hbm, idx_v, vals_v, sem):
        wid = lax.axis_index("s") * info.num_cores + lax.axis_index("c")
        base = wid * npw
        pltpu.sync_copy(idx_hbm.at[pl.ds(base, npw)], idx_v)

        def outer(j0, carry):
            descs = [
                pltpu.async_copy(
                    means_hbm.at[idx_v.at[pl.ds((j0 * k + b) * _CHUNK,
                                                _CHUNK)]],
                    vals_v.at[pl.ds((j0 * k + b) * _CHUNK, _CHUNK)],
                    sem)
                for b in range(k)
            ]
            for d in descs:
                d.wait()
            return carry

        lax.fori_loop(0, chunks // k, outer, 0)
        pltpu.sync_copy(vals_v, out_hbm.at[pl.ds(base, npw)])

    return gather_kernel(means, idx1d)


def _norm_body(x_ref, o_ref):
    x = x_ref[...]
    ss = jnp.sum(x * x, axis=1, keepdims=True)
    o_ref[...] = x / jnp.sqrt(ss)


def _normalize(vals):
    seqs, seq_len = vals.shape
    blk = 4096
    return pl.pallas_call(
        _norm_body,
        grid=(seqs // blk,),
        in_specs=[pl.BlockSpec((blk, seq_len), lambda i: (i, 0))],
        out_specs=pl.BlockSpec((blk, seq_len), lambda i: (i, 0)),
        out_shape=jax.ShapeDtypeStruct((seqs, seq_len), jnp.float32),
    )(vals)


def kernel(anchor_input_ids, positive_input_ids, negative_input_ids,
           embedding_table):
    batch, seq = anchor_input_ids.shape
    num_neg = negative_input_ids.shape[1]

    means = _row_means(embedding_table)
    ids = jnp.concatenate([
        anchor_input_ids.reshape(-1),
        positive_input_ids.reshape(-1),
        negative_input_ids.reshape(-1),
    ]).astype(jnp.int32)
    vals = _gather_means(means, ids)
    out = _normalize(vals.reshape(-1, seq))

    anchor = out[:batch].reshape(batch, seq, 1)
    positive = out[batch:2 * batch].reshape(batch, seq, 1)
    negative = out[2 * batch:].reshape(batch, num_neg, seq)
    return (anchor, positive, negative)


# speedup vs baseline: 5.3215x; 5.3215x over previous
import functools

import jax
import jax.numpy as jnp
from jax import lax
from jax.experimental import pallas as pl
from jax.experimental.pallas import tpu as pltpu
from jax.experimental.pallas import tpu_sc as plsc

_DIM = 32
_CHUNK = 128
_MBLK = 65536


def _row_mean_body(x_ref, o_ref):
    o_ref[...] = jnp.sum(x_ref[...], axis=0) * (1.0 / _DIM)


def _row_means(table_t):
    rows = table_t.shape[1]
    grid = (rows + _MBLK - 1) // _MBLK
    return pl.pallas_call(
        _row_mean_body,
        grid=(grid,),
        in_specs=[pl.BlockSpec((_DIM, _MBLK), lambda i: (0, i))],
        out_specs=pl.BlockSpec((_MBLK,), lambda i: (i,)),
        out_shape=jax.ShapeDtypeStruct((grid * _MBLK,), jnp.float32),
    )(table_t)


def _gather_means(means, idx1d):
    info = plsc.get_sparse_core_info()
    nw = info.num_cores * info.num_subcores
    n = idx1d.shape[0]
    npw = n // nw
    chunks = npw // _CHUNK
    k = 10
    mesh = plsc.VectorSubcoreMesh(core_axis_name="c", subcore_axis_name="s")

    @functools.partial(
        pl.kernel, mesh=mesh,
        out_type=jax.ShapeDtypeStruct((n,), jnp.float32),
        scratch_types=[
            pltpu.VMEM((npw,), jnp.int32),
            pltpu.VMEM((npw,), jnp.float32),
            pltpu.SemaphoreType.DMA,
        ],
    )
    def gather_kernel(means_hbm, idx_hbm, out_hbm, idx_v, vals_v, sem):
        wid = lax.axis_index("s") * info.num_cores + lax.axis_index("c")
        base = wid * npw
        pltpu.sync_copy(idx_hbm.at[pl.ds(base, npw)], idx_v)

        def outer(j0, carry):
            descs = [
                pltpu.async_copy(
                    means_hbm.at[idx_v.at[pl.ds((j0 * k + b) * _CHUNK,
                                                _CHUNK)]],
                    vals_v.at[pl.ds((j0 * k + b) * _CHUNK, _CHUNK)],
                    sem)
                for b in range(k)
            ]
            for d in descs:
                d.wait()
            return carry

        lax.fori_loop(0, chunks // k, outer, 0)
        pltpu.sync_copy(vals_v, out_hbm.at[pl.ds(base, npw)])

    return gather_kernel(means, idx1d)


def _norm_body(a_ref, p_ref, n_ref, oa_ref, op_ref, on_ref):
    for x_ref, o_ref in ((a_ref, oa_ref), (p_ref, op_ref), (n_ref, on_ref)):
        x = x_ref[...]
        ss = jnp.sum(x * x, axis=0, keepdims=True)
        o_ref[...] = x / jnp.sqrt(ss)


def _normalize(va, vp, vn):
    seq = va.shape[0]
    ca, cn = va.shape[1], vn.shape[1]
    grid = 8
    ba, bn = ca // grid, cn // grid
    return pl.pallas_call(
        _norm_body,
        grid=(grid,),
        in_specs=[pl.BlockSpec((seq, ba), lambda i: (0, i)),
                  pl.BlockSpec((seq, ba), lambda i: (0, i)),
                  pl.BlockSpec((seq, bn), lambda i: (0, i))],
        out_specs=[pl.BlockSpec((seq, ba), lambda i: (0, i)),
                   pl.BlockSpec((seq, ba), lambda i: (0, i)),
                   pl.BlockSpec((seq, bn), lambda i: (0, i))],
        out_shape=[jax.ShapeDtypeStruct((seq, ca), jnp.float32),
                   jax.ShapeDtypeStruct((seq, ca), jnp.float32),
                   jax.ShapeDtypeStruct((seq, cn), jnp.float32)],
    )(va, vp, vn)


def kernel(anchor_input_ids, positive_input_ids, negative_input_ids,
           embedding_table):
    batch, seq = anchor_input_ids.shape
    num_neg = negative_input_ids.shape[1]
    na, nn = batch * seq, batch * num_neg * seq

    means = _row_means(embedding_table.T)
    ids = jnp.concatenate([
        anchor_input_ids.T.reshape(-1),
        positive_input_ids.T.reshape(-1),
        negative_input_ids.transpose(2, 1, 0).reshape(-1),
    ]).astype(jnp.int32)
    vals = _gather_means(means, ids)

    va = vals[:na].reshape(seq, batch)
    vp = vals[na:2 * na].reshape(seq, batch)
    vn = vals[2 * na:].reshape(seq, num_neg * batch)
    oa, op_, on = _normalize(va, vp, vn)

    anchor = oa.T.reshape(batch, seq, 1)
    positive = op_.T.reshape(batch, seq, 1)
    negative = on.reshape(seq, num_neg, batch).transpose(2, 1, 0)
    return (anchor, positive, negative)


# speedup vs baseline: 5.6770x; 1.0668x over previous
import functools

import jax
import jax.numpy as jnp
from jax import lax
from jax.experimental import pallas as pl
from jax.experimental.pallas import tpu as pltpu
from jax.experimental.pallas import tpu_sc as plsc

_DIM = 32
_CHUNK = 128
_MBLK = 65536


def _row_mean_body(x_ref, o_ref):
    o_ref[...] = jnp.sum(x_ref[...], axis=0) * (1.0 / _DIM)


def _row_means(table_t):
    rows = table_t.shape[1]
    grid = (rows + _MBLK - 1) // _MBLK
    return pl.pallas_call(
        _row_mean_body,
        grid=(grid,),
        in_specs=[pl.BlockSpec((_DIM, _MBLK), lambda i: (0, i))],
        out_specs=pl.BlockSpec((_MBLK,), lambda i: (i,)),
        out_shape=jax.ShapeDtypeStruct((grid * _MBLK,), jnp.float32),
    )(table_t)


def _gather_means(means, idx1d):
    info = plsc.get_sparse_core_info()
    nw = info.num_cores * info.num_subcores
    n = idx1d.shape[0]
    npw = n // nw
    chunks = npw // _CHUNK
    k = 20
    mesh = plsc.VectorSubcoreMesh(core_axis_name="c", subcore_axis_name="s")

    @functools.partial(
        pl.kernel, mesh=mesh,
        out_type=jax.ShapeDtypeStruct((n,), jnp.float32),
        scratch_types=[
            pltpu.VMEM((npw,), jnp.int32),
            pltpu.VMEM((npw,), jnp.float32),
            pltpu.SemaphoreType.DMA,
        ],
    )
    def gather_kernel(means_hbm, idx_hbm, out_hbm, idx_v, vals_v, sem):
        wid = lax.axis_index("s") * info.num_cores + lax.axis_index("c")
        base = wid * npw
        pltpu.sync_copy(idx_hbm.at[pl.ds(base, npw)], idx_v)

        def outer(j0, carry):
            descs = [
                pltpu.async_copy(
                    means_hbm.at[idx_v.at[pl.ds((j0 * k + b) * _CHUNK,
                                                _CHUNK)]],
                    vals_v.at[pl.ds((j0 * k + b) * _CHUNK, _CHUNK)],
                    sem)
                for b in range(k)
            ]
            for d in descs:
                d.wait()
            return carry

        lax.fori_loop(0, chunks // k, outer, 0)
        pltpu.sync_copy(vals_v, out_hbm.at[pl.ds(base, npw)])

    return gather_kernel(means, idx1d)


def _norm_body(a_ref, p_ref, n_ref, oa_ref, op_ref, on_ref):
    for x_ref, o_ref in ((a_ref, oa_ref), (p_ref, op_ref), (n_ref, on_ref)):
        x = x_ref[...]
        ss = jnp.sum(x * x, axis=0, keepdims=True)
        o_ref[...] = x / jnp.sqrt(ss)


def _normalize(va, vp, vn):
    seq = va.shape[0]
    ca, cn = va.shape[1], vn.shape[1]
    grid = 8
    ba, bn = ca // grid, cn // grid
    return pl.pallas_call(
        _norm_body,
        grid=(grid,),
        in_specs=[pl.BlockSpec((seq, ba), lambda i: (0, i)),
                  pl.BlockSpec((seq, ba), lambda i: (0, i)),
                  pl.BlockSpec((seq, bn), lambda i: (0, i))],
        out_specs=[pl.BlockSpec((seq, ba), lambda i: (0, i)),
                   pl.BlockSpec((seq, ba), lambda i: (0, i)),
                   pl.BlockSpec((seq, bn), lambda i: (0, i))],
        out_shape=[jax.ShapeDtypeStruct((seq, ca), jnp.float32),
                   jax.ShapeDtypeStruct((seq, ca), jnp.float32),
                   jax.ShapeDtypeStruct((seq, cn), jnp.float32)],
    )(va, vp, vn)


def kernel(anchor_input_ids, positive_input_ids, negative_input_ids,
           embedding_table):
    batch, seq = anchor_input_ids.shape
    num_neg = negative_input_ids.shape[1]
    na, nn = batch * seq, batch * num_neg * seq

    means = _row_means(embedding_table.T)
    ids = jnp.concatenate([
        anchor_input_ids.T.reshape(-1),
        positive_input_ids.T.reshape(-1),
        negative_input_ids.transpose(2, 1, 0).reshape(-1),
    ]).astype(jnp.int32)
    vals = _gather_means(means, ids)

    va = vals[:na].reshape(seq, batch)
    vp = vals[na:2 * na].reshape(seq, batch)
    vn = vals[2 * na:].reshape(seq, num_neg * batch)
    oa, op_, on = _normalize(va, vp, vn)

    anchor = oa.T.reshape(batch, seq, 1)
    positive = op_.T.reshape(batch, seq, 1)
    negative = on.reshape(seq, num_neg, batch).transpose(2, 1, 0)
    return (anchor, positive, negative)


# speedup vs baseline: 6.0638x; 1.0681x over previous
import functools

import jax
import jax.numpy as jnp
from jax import lax
from jax.experimental import pallas as pl
from jax.experimental.pallas import tpu as pltpu
from jax.experimental.pallas import tpu_sc as plsc

_DIM = 32
_CHUNK = 128
_MBLK = 65536


def _row_mean_body(x_ref, o_ref):
    o_ref[...] = jnp.sum(x_ref[...], axis=0) * (1.0 / _DIM)


def _row_means(table_t):
    rows = table_t.shape[1]
    grid = (rows + _MBLK - 1) // _MBLK
    return pl.pallas_call(
        _row_mean_body,
        grid=(grid,),
        in_specs=[pl.BlockSpec((_DIM, _MBLK), lambda i: (0, i))],
        out_specs=pl.BlockSpec((_MBLK,), lambda i: (i,)),
        out_shape=jax.ShapeDtypeStruct((grid * _MBLK,), jnp.float32),
    )(table_t)


def _gather_means(means, idx1d):
    info = plsc.get_sparse_core_info()
    nw = info.num_cores * info.num_subcores
    n = idx1d.shape[0]
    npw = n // nw
    chunks = npw // _CHUNK
    k = 20
    mesh = plsc.VectorSubcoreMesh(core_axis_name="c", subcore_axis_name="s")

    @functools.partial(
        pl.kernel, mesh=mesh,
        out_type=jax.ShapeDtypeStruct((n,), jnp.float32),
        scratch_types=[
            pltpu.VMEM((npw,), jnp.int32),
            pltpu.VMEM((npw,), jnp.float32),
            pltpu.SemaphoreType.DMA,
        ],
    )
    def gather_kernel(means_hbm, idx_hbm, out_hbm, idx_v, vals_v, sem):
        wid = lax.axis_index("s") * info.num_cores + lax.axis_index("c")
        base = wid * npw
        pltpu.sync_copy(idx_hbm.at[pl.ds(base, npw)], idx_v)
        pltpu.async_copy(means_hbm.at[idx_v], vals_v, sem).wait()
        pltpu.sync_copy(vals_v, out_hbm.at[pl.ds(base, npw)])

    return gather_kernel(means, idx1d)


def _norm_body(a_ref, p_ref, n_ref, oa_ref, op_ref, on_ref):
    for x_ref, o_ref in ((a_ref, oa_ref), (p_ref, op_ref), (n_ref, on_ref)):
        x = x_ref[...]
        ss = jnp.sum(x * x, axis=0, keepdims=True)
        o_ref[...] = x / jnp.sqrt(ss)


def _normalize(va, vp, vn):
    seq = va.shape[0]
    ca, cn = va.shape[1], vn.shape[1]
    grid = 8
    ba, bn = ca // grid, cn // grid
    return pl.pallas_call(
        _norm_body,
        grid=(grid,),
        in_specs=[pl.BlockSpec((seq, ba), lambda i: (0, i)),
                  pl.BlockSpec((seq, ba), lambda i: (0, i)),
                  pl.BlockSpec((seq, bn), lambda i: (0, i))],
        out_specs=[pl.BlockSpec((seq, ba), lambda i: (0, i)),
                   pl.BlockSpec((seq, ba), lambda i: (0, i)),
                   pl.BlockSpec((seq, bn), lambda i: (0, i))],
        out_shape=[jax.ShapeDtypeStruct((seq, ca), jnp.float32),
                   jax.ShapeDtypeStruct((seq, ca), jnp.float32),
                   jax.ShapeDtypeStruct((seq, cn), jnp.float32)],
    )(va, vp, vn)


def kernel(anchor_input_ids, positive_input_ids, negative_input_ids,
           embedding_table):
    batch, seq = anchor_input_ids.shape
    num_neg = negative_input_ids.shape[1]
    na, nn = batch * seq, batch * num_neg * seq

    means = _row_means(embedding_table.T)
    ids = jnp.concatenate([
        anchor_input_ids.T.reshape(-1),
        positive_input_ids.T.reshape(-1),
        negative_input_ids.transpose(2, 1, 0).reshape(-1),
    ]).astype(jnp.int32)
    vals = _gather_means(means, ids)

    va = vals[:na].reshape(seq, batch)
    vp = vals[na:2 * na].reshape(seq, batch)
    vn = vals[2 * na:].reshape(seq, num_neg * batch)
    oa, op_, on = _normalize(va, vp, vn)

    anchor = oa.T.reshape(batch, seq, 1)
    positive = op_.T.reshape(batch, seq, 1)
    negative = on.reshape(seq, num_neg, batch).transpose(2, 1, 0)
    return (anchor, positive, negative)


# speedup vs baseline: 7.0697x; 1.1659x over previous
import functools

import jax
import jax.numpy as jnp
from jax import lax
from jax.experimental import pallas as pl
from jax.experimental.pallas import tpu as pltpu
from jax.experimental.pallas import tpu_sc as plsc

_DIM = 32
_CHUNK = 128
_MBLK = 65536


def _row_mean_body(x_ref, o_ref):
    o_ref[...] = jnp.sum(x_ref[...], axis=0) * (1.0 / _DIM)


def _row_means(table_t):
    rows = table_t.shape[1]
    grid = (rows + _MBLK - 1) // _MBLK
    return pl.pallas_call(
        _row_mean_body,
        grid=(grid,),
        in_specs=[pl.BlockSpec((_DIM, _MBLK), lambda i: (0, i))],
        out_specs=pl.BlockSpec((_MBLK,), lambda i: (i,)),
        out_shape=jax.ShapeDtypeStruct((grid * _MBLK,), jnp.float32),
    )(table_t)


def _gather_means(means, idx1d):
    info = plsc.get_sparse_core_info()
    nw = info.num_cores * info.num_subcores
    n = idx1d.shape[0]
    npw = n // nw
    chunks = npw // _CHUNK
    k = 20
    mesh = plsc.VectorSubcoreMesh(core_axis_name="c", subcore_axis_name="s")

    @functools.partial(
        pl.kernel, mesh=mesh,
        out_type=jax.ShapeDtypeStruct((n,), jnp.float32),
        scratch_types=[
            pltpu.VMEM((npw,), jnp.int32),
            pltpu.VMEM((npw,), jnp.float32),
            pltpu.SemaphoreType.DMA,
        ],
    )
    def gather_kernel(means_hbm, idx_hbm, out_hbm, idx_v, vals_v, sem):
        wid = lax.axis_index("s") * info.num_cores + lax.axis_index("c")
        base = wid * npw
        pltpu.sync_copy(idx_hbm.at[pl.ds(base, npw)], idx_v)
        pltpu.async_copy(means_hbm.at[idx_v], vals_v, sem).wait()
        pltpu.sync_copy(vals_v, out_hbm.at[pl.ds(base, npw)])

    return gather_kernel(means, idx1d)


def _norm_body(a_ref, p_ref, n_ref, oa_ref, op_ref, on_ref):
    for x_ref, o_ref in ((a_ref, oa_ref), (p_ref, op_ref), (n_ref, on_ref)):
        x = x_ref[...]
        ss = jnp.sum(x * x, axis=0, keepdims=True)
        o_ref[...] = x / jnp.sqrt(ss)


def _normalize(va, vp, vn):
    seq = va.shape[0]
    ca, cn = va.shape[2], vn.shape[2]
    grid = 8
    ba, bn = ca // grid, cn // grid
    return pl.pallas_call(
        _norm_body,
        grid=(grid,),
        in_specs=[pl.BlockSpec((seq, 1, ba), lambda i: (0, 0, i)),
                  pl.BlockSpec((seq, 1, ba), lambda i: (0, 0, i)),
                  pl.BlockSpec((seq, 1, bn), lambda i: (0, 0, i))],
        out_specs=[pl.BlockSpec((seq, 1, ba), lambda i: (0, 0, i)),
                   pl.BlockSpec((seq, 1, ba), lambda i: (0, 0, i)),
                   pl.BlockSpec((seq, 1, bn), lambda i: (0, 0, i))],
        out_shape=[jax.ShapeDtypeStruct((seq, 1, ca), jnp.float32),
                   jax.ShapeDtypeStruct((seq, 1, ca), jnp.float32),
                   jax.ShapeDtypeStruct((seq, 1, cn), jnp.float32)],
    )(va, vp, vn)


def kernel(anchor_input_ids, positive_input_ids, negative_input_ids,
           embedding_table):
    batch, seq = anchor_input_ids.shape
    num_neg = negative_input_ids.shape[1]
    na, nn = batch * seq, batch * num_neg * seq

    means = _row_means(embedding_table.T)
    nt = negative_input_ids.transpose(2, 1, 0)
    nt = nt.reshape(seq, num_neg, batch // 128, 128).transpose(0, 2, 1, 3)
    ids = jnp.concatenate([
        anchor_input_ids.T.reshape(-1),
        positive_input_ids.T.reshape(-1),
        nt.reshape(-1),
    ]).astype(jnp.int32)
    vals = _gather_means(means, ids)

    va = vals[:na].reshape(seq, 1, batch)
    vp = vals[na:2 * na].reshape(seq, 1, batch)
    vn = vals[2 * na:].reshape(seq, 1, num_neg * batch)
    oa, op_, on = _normalize(va, vp, vn)

    anchor = oa.transpose(2, 0, 1)
    positive = op_.transpose(2, 0, 1)
    negative = (on.reshape(seq, batch // 128, num_neg, 128)
                .transpose(1, 3, 2, 0).reshape(batch, num_neg, seq))
    return (anchor, positive, negative)
